# baseline (speedup 1.0000x reference)
import functools

import jax
import jax.numpy as jnp
from jax.experimental import pallas as pl
from jax.experimental.pallas import tpu as pltpu

B, N = 8, 4096
NPOINT = [1024, 256, 64, 16]
NSAMPLE = [4, 8, 16, 32]
RES = [32, 16, 8, 4]


def _square_distance(src, dst):
    return (jnp.sum(src ** 2, -1)[:, :, None] + jnp.sum(dst ** 2, -1)[:, None, :]
            - 2.0 * jnp.einsum('bnd,bmd->bnm', src, dst))


def _index_points(points, idx):
    return jax.vmap(lambda p, i: p[i])(points, idx)


def _farthest_point_sample(xyz, npoint):
    def single(x):
        n = x.shape[0]
        def body(i, state):
            dist, idxs = state
            cur = x[idxs[i]]
            d = jnp.sum((x - cur[None, :]) ** 2, -1)
            dist = jnp.minimum(dist, d)
            idxs = idxs.at[i + 1].set(jnp.argmax(dist).astype(jnp.int32))
            return (dist, idxs)
        idxs0 = jnp.zeros((npoint,), jnp.int32)
        dist0 = jnp.full((n,), 1e10, jnp.float32)
        _, idxs = jax.lax.fori_loop(0, npoint - 1, body, (dist0, idxs0))
        return idxs
    return jax.vmap(single)(xyz)


def _compute_fps(pts):
    pts = jax.lax.stop_gradient(pts)
    idx_list = []
    cur = pts
    for npt in NPOINT:
        i = _farthest_point_sample(cur, npt)
        idx_list.append(i)
        cur = _index_points(cur, i)
    return idx_list


def _knn(nsample, xyz, new_xyz):
    d = _square_distance(jax.lax.stop_gradient(new_xyz), jax.lax.stop_gradient(xyz))
    _, idx = jax.lax.top_k(-d, nsample)
    return idx


def _batch_norm(x, g, b):
    axes = tuple(range(x.ndim - 1))
    mu = jnp.mean(x, axis=axes, keepdims=True)
    var = jnp.var(x, axis=axes, keepdims=True)
    return (x - mu) / jnp.sqrt(var + 1e-5) * g + b


def _ri_features(g_xyz, g_norm, new_xyz, new_norm):
    vec = g_xyz - new_xyz[:, :, None, :]
    dist = jnp.linalg.norm(vec, axis=-1, keepdims=True)
    u = vec / (dist + 1e-8)
    m = jnp.mean(g_xyz, axis=2, keepdims=True)
    vec_m = g_xyz - m
    dist_m = jnp.linalg.norm(vec_m, axis=-1, keepdims=True)
    um = vec_m / (dist_m + 1e-8)
    np_ = new_norm[:, :, None, :]
    a1 = jnp.sum(u * np_, -1, keepdims=True)
    a2 = jnp.sum(u * g_norm, -1, keepdims=True)
    a3 = jnp.sum(np_ * g_norm, -1, keepdims=True)
    a4 = jnp.sum(um * g_norm, -1, keepdims=True)
    a5 = jnp.sum(um * np_, -1, keepdims=True)
    return jnp.concatenate([dist, dist_m, a1, a2, a3, a4, a5], -1)


def _set_abstraction(xyz, norm, points, feature, fps_idx, nsample, p):
    new_xyz = _index_points(xyz, fps_idx)
    new_norm = _index_points(norm, fps_idx)
    new_feature = _index_points(feature, fps_idx)
    idx = _knn(nsample, xyz, new_xyz)
    g_xyz = _index_points(xyz, idx)
    g_norm = _index_points(norm, idx)
    ri = _ri_features(g_xyz, g_norm, new_xyz, new_norm)
    lifted = jax.nn.relu(_batch_norm(ri @ p['prev_w'], p['prev_g'], p['prev_b']))
    if points is not None:
        feats = jnp.concatenate([lifted, _index_points(points, idx)], -1)
    else:
        feats = lifted
    out = jax.nn.relu(_batch_norm(feats @ p['w'], p['g'], p['b']))
    return new_xyz, new_norm, jnp.max(out, axis=2), new_feature


def _feature_propagation(xyz1, xyz2, points1, points2, p):
    d = _square_distance(xyz1, xyz2)
    negd, idx = jax.lax.top_k(-d, 3)
    d3 = jnp.maximum(-negd, 0.0)
    w = 1.0 / (d3 + 1e-8)
    w = w / jnp.sum(w, -1, keepdims=True)
    neigh = _index_points(points2, idx)
    interp = jnp.sum(neigh * w[..., None], axis=2)
    feats = interp if points1 is None else jnp.concatenate([interp, points1], -1)
    out = jax.nn.relu(_batch_norm(feats @ p['w1'], p['g1'], p['b1']))
    if 'w2' in p:
        out = jax.nn.relu(_batch_norm(out @ p['w2'], p['g2'], p['b2']))
    return out


def _voxelize(xyz, feats, r):
    mn = jnp.min(xyz, axis=1, keepdims=True)
    mx = jnp.max(xyz, axis=1, keepdims=True)
    nc = (xyz - mn) / (mx - mn + 1e-8)
    coords = jnp.clip((nc * r).astype(jnp.int32), 0, r - 1)
    flat = coords[..., 0] * (r * r) + coords[..., 1] * r + coords[..., 2]
    def single(f, fl):
        s = jax.ops.segment_sum(f, fl, num_segments=r * r * r)
        c = jax.ops.segment_sum(jnp.ones(fl.shape, jnp.float32), fl, num_segments=r * r * r)
        return s / jnp.maximum(c, 1.0)[:, None]
    vox = jax.vmap(single)(feats, flat)
    Bv = vox.shape[0]
    C = vox.shape[2]
    return jnp.transpose(vox, (0, 2, 1)).reshape(Bv, C, r, r, r)


def _head_kernel(point_ref, w1_ref, bias1_ref, g1_ref, b1_ref, w2_ref, bias2_ref,
                 out_ref):
    x = point_ref[...].reshape(B * N, -1)
    pre = jnp.dot(x, w1_ref[...], preferred_element_type=jnp.float32) + bias1_ref[...]
    mu = jnp.mean(pre, axis=0, keepdims=True)
    var = jnp.mean((pre - mu) ** 2, axis=0, keepdims=True)
    feat = jax.nn.relu((pre - mu) / jnp.sqrt(var + 1e-5) * g1_ref[...] + b1_ref[...])
    y = jax.nn.sigmoid(
        jnp.dot(feat, w2_ref[...], preferred_element_type=jnp.float32) + bias2_ref[...])
    out_ref[...] = y.reshape(B, N, 1)


def _head(point, h):
    return pl.pallas_call(
        _head_kernel,
        out_shape=jax.ShapeDtypeStruct((B, N, 1), jnp.float32),
    )(point, h['w1'], h['bias1'].reshape(1, -1), h['g1'].reshape(1, -1),
      h['b1'].reshape(1, -1), h['w2'], h['bias2'].reshape(1, -1))


def kernel(xyz, feature, params):
    norm = xyz[:, :, 3:]
    pts = xyz[:, :, :3]
    fps_list = _compute_fps(pts)
    VPoints = []
    l0_xyz, l0_norm, l0_points, feature = _set_abstraction(
        pts, norm, None, feature, fps_list[0], NSAMPLE[0], params['sa0'])
    VPoints.append(_voxelize(l0_xyz, l0_points, RES[0]))
    l1_xyz, l1_norm, l1_points, feature = _set_abstraction(
        l0_xyz, l0_norm, l0_points, feature, fps_list[1], NSAMPLE[1], params['sa1'])
    VPoints.append(_voxelize(l1_xyz, l1_points, RES[1]))
    l2_xyz, l2_norm, l2_points, feature = _set_abstraction(
        l1_xyz, l1_norm, l1_points, feature, fps_list[2], NSAMPLE[2], params['sa2'])
    VPoints.append(_voxelize(l2_xyz, l2_points, RES[2]))
    l3_xyz, l3_norm, l3_points, feature = _set_abstraction(
        l2_xyz, l2_norm, l2_points, feature, fps_list[3], NSAMPLE[3], params['sa3'])
    VPoints.append(_voxelize(l3_xyz, l3_points, RES[3]))
    l2_points = _feature_propagation(l2_xyz, l3_xyz, l2_points, l3_points, params['fp3'])
    VPoints.append(_voxelize(l2_xyz, l2_points, RES[2]))
    l1_points = _feature_propagation(l1_xyz, l2_xyz, l1_points, l2_points, params['fp2'])
    VPoints.append(_voxelize(l1_xyz, l1_points, RES[1]))
    l0_points = _feature_propagation(l0_xyz, l1_xyz, l0_points, l1_points, params['fp1'])
    VPoints.append(_voxelize(l0_xyz, l0_points, RES[0]))
    point = _feature_propagation(pts, l0_xyz, None, l0_points, params['fp0'])
    x = _head(point, params['head'])
    return (x,) + tuple(VPoints)


# speedup vs baseline: 1.7199x; 1.7199x over previous
import functools

import jax
import jax.numpy as jnp
from jax.experimental import pallas as pl
from jax.experimental.pallas import tpu as pltpu

B, N = 8, 4096
NPOINT = [1024, 256, 64, 16]
NSAMPLE = [4, 8, 16, 32]
RES = [32, 16, 8, 4]


def _square_distance(src, dst):
    return (jnp.sum(src ** 2, -1)[:, :, None] + jnp.sum(dst ** 2, -1)[:, None, :]
            - 2.0 * jnp.einsum('bnd,bmd->bnm', src, dst))


def _index_points(points, idx):
    return jax.vmap(lambda p, i: p[i])(points, idx)


def _fps_kernel(npoint, xs_ref, idx_ref):
    _, Bb, n = xs_ref.shape
    x0 = xs_ref[0]
    x1 = xs_ref[1]
    x2 = xs_ref[2]
    lane = jax.lax.broadcasted_iota(jnp.int32, (Bb, n), 1)
    col = jax.lax.broadcasted_iota(jnp.int32, (Bb, npoint), 1)

    def body(i, state):
        dist, idxs, amax = state
        h = jnp.where(lane == amax[:, None], 1.0, 0.0)
        c0 = jnp.sum(x0 * h, axis=1)[:, None]
        c1 = jnp.sum(x1 * h, axis=1)[:, None]
        c2 = jnp.sum(x2 * h, axis=1)[:, None]
        d0 = x0 - c0
        d1 = x1 - c1
        d2 = x2 - c2
        d = (d0 * d0 + d1 * d1) + d2 * d2
        dist = jnp.minimum(dist, d)
        amax = jnp.argmax(dist, axis=1).astype(jnp.int32)
        idxs = jnp.where(col == i + 1, amax[:, None], idxs)
        return (dist, idxs, amax)

    dist0 = jnp.full((Bb, n), 1e10, jnp.float32)
    idxs0 = jnp.zeros((Bb, npoint), jnp.int32)
    amax0 = jnp.zeros((Bb,), jnp.int32)
    _, idxs, _ = jax.lax.fori_loop(0, npoint - 1, body, (dist0, idxs0, amax0))
    idx_ref[...] = idxs


def _farthest_point_sample(xyz, npoint):
    b, n, _ = xyz.shape
    xs = jnp.transpose(xyz, (2, 0, 1))
    return pl.pallas_call(
        functools.partial(_fps_kernel, npoint),
        out_shape=jax.ShapeDtypeStruct((b, npoint), jnp.int32),
    )(xs)


def _compute_fps(pts):
    pts = jax.lax.stop_gradient(pts)
    idx_list = []
    cur = pts
    for npt in NPOINT:
        i = _farthest_point_sample(cur, npt)
        idx_list.append(i)
        cur = _index_points(cur, i)
    return idx_list


def _knn(nsample, xyz, new_xyz):
    d = _square_distance(jax.lax.stop_gradient(new_xyz), jax.lax.stop_gradient(xyz))
    _, idx = jax.lax.top_k(-d, nsample)
    return idx


def _batch_norm(x, g, b):
    axes = tuple(range(x.ndim - 1))
    mu = jnp.mean(x, axis=axes, keepdims=True)
    var = jnp.var(x, axis=axes, keepdims=True)
    return (x - mu) / jnp.sqrt(var + 1e-5) * g + b


def _ri_features(g_xyz, g_norm, new_xyz, new_norm):
    vec = g_xyz - new_xyz[:, :, None, :]
    dist = jnp.linalg.norm(vec, axis=-1, keepdims=True)
    u = vec / (dist + 1e-8)
    m = jnp.mean(g_xyz, axis=2, keepdims=True)
    vec_m = g_xyz - m
    dist_m = jnp.linalg.norm(vec_m, axis=-1, keepdims=True)
    um = vec_m / (dist_m + 1e-8)
    np_ = new_norm[:, :, None, :]
    a1 = jnp.sum(u * np_, -1, keepdims=True)
    a2 = jnp.sum(u * g_norm, -1, keepdims=True)
    a3 = jnp.sum(np_ * g_norm, -1, keepdims=True)
    a4 = jnp.sum(um * g_norm, -1, keepdims=True)
    a5 = jnp.sum(um * np_, -1, keepdims=True)
    return jnp.concatenate([dist, dist_m, a1, a2, a3, a4, a5], -1)


def _set_abstraction(xyz, norm, points, feature, fps_idx, nsample, p):
    new_xyz = _index_points(xyz, fps_idx)
    new_norm = _index_points(norm, fps_idx)
    new_feature = _index_points(feature, fps_idx)
    idx = _knn(nsample, xyz, new_xyz)
    g_xyz = _index_points(xyz, idx)
    g_norm = _index_points(norm, idx)
    ri = _ri_features(g_xyz, g_norm, new_xyz, new_norm)
    lifted = jax.nn.relu(_batch_norm(ri @ p['prev_w'], p['prev_g'], p['prev_b']))
    if points is not None:
        feats = jnp.concatenate([lifted, _index_points(points, idx)], -1)
    else:
        feats = lifted
    out = jax.nn.relu(_batch_norm(feats @ p['w'], p['g'], p['b']))
    return new_xyz, new_norm, jnp.max(out, axis=2), new_feature


def _feature_propagation(xyz1, xyz2, points1, points2, p):
    d = _square_distance(xyz1, xyz2)
    negd, idx = jax.lax.top_k(-d, 3)
    d3 = jnp.maximum(-negd, 0.0)
    w = 1.0 / (d3 + 1e-8)
    w = w / jnp.sum(w, -1, keepdims=True)
    neigh = _index_points(points2, idx)
    interp = jnp.sum(neigh * w[..., None], axis=2)
    feats = interp if points1 is None else jnp.concatenate([interp, points1], -1)
    out = jax.nn.relu(_batch_norm(feats @ p['w1'], p['g1'], p['b1']))
    if 'w2' in p:
        out = jax.nn.relu(_batch_norm(out @ p['w2'], p['g2'], p['b2']))
    return out


def _voxelize(xyz, feats, r):
    mn = jnp.min(xyz, axis=1, keepdims=True)
    mx = jnp.max(xyz, axis=1, keepdims=True)
    nc = (xyz - mn) / (mx - mn + 1e-8)
    coords = jnp.clip((nc * r).astype(jnp.int32), 0, r - 1)
    flat = coords[..., 0] * (r * r) + coords[..., 1] * r + coords[..., 2]
    def single(f, fl):
        s = jax.ops.segment_sum(f, fl, num_segments=r * r * r)
        c = jax.ops.segment_sum(jnp.ones(fl.shape, jnp.float32), fl, num_segments=r * r * r)
        return s / jnp.maximum(c, 1.0)[:, None]
    vox = jax.vmap(single)(feats, flat)
    Bv = vox.shape[0]
    C = vox.shape[2]
    return jnp.transpose(vox, (0, 2, 1)).reshape(Bv, C, r, r, r)


def _head_kernel(point_ref, w1_ref, bias1_ref, g1_ref, b1_ref, w2_ref, bias2_ref,
                 out_ref):
    x = point_ref[...].reshape(B * N, -1)
    pre = jnp.dot(x, w1_ref[...], preferred_element_type=jnp.float32) + bias1_ref[...]
    mu = jnp.mean(pre, axis=0, keepdims=True)
    var = jnp.mean((pre - mu) ** 2, axis=0, keepdims=True)
    feat = jax.nn.relu((pre - mu) / jnp.sqrt(var + 1e-5) * g1_ref[...] + b1_ref[...])
    y = jax.nn.sigmoid(
        jnp.dot(feat, w2_ref[...], preferred_element_type=jnp.float32) + bias2_ref[...])
    out_ref[...] = y.reshape(B, N, 1)


def _head(point, h):
    return pl.pallas_call(
        _head_kernel,
        out_shape=jax.ShapeDtypeStruct((B, N, 1), jnp.float32),
    )(point, h['w1'], h['bias1'].reshape(1, -1), h['g1'].reshape(1, -1),
      h['b1'].reshape(1, -1), h['w2'], h['bias2'].reshape(1, -1))


def kernel(xyz, feature, params):
    norm = xyz[:, :, 3:]
    pts = xyz[:, :, :3]
    fps_list = _compute_fps(pts)
    VPoints = []
    l0_xyz, l0_norm, l0_points, feature = _set_abstraction(
        pts, norm, None, feature, fps_list[0], NSAMPLE[0], params['sa0'])
    VPoints.append(_voxelize(l0_xyz, l0_points, RES[0]))
    l1_xyz, l1_norm, l1_points, feature = _set_abstraction(
        l0_xyz, l0_norm, l0_points, feature, fps_list[1], NSAMPLE[1], params['sa1'])
    VPoints.append(_voxelize(l1_xyz, l1_points, RES[1]))
    l2_xyz, l2_norm, l2_points, feature = _set_abstraction(
        l1_xyz, l1_norm, l1_points, feature, fps_list[2], NSAMPLE[2], params['sa2'])
    VPoints.append(_voxelize(l2_xyz, l2_points, RES[2]))
    l3_xyz, l3_norm, l3_points, feature = _set_abstraction(
        l2_xyz, l2_norm, l2_points, feature, fps_list[3], NSAMPLE[3], params['sa3'])
    VPoints.append(_voxelize(l3_xyz, l3_points, RES[3]))
    l2_points = _feature_propagation(l2_xyz, l3_xyz, l2_points, l3_points, params['fp3'])
    VPoints.append(_voxelize(l2_xyz, l2_points, RES[2]))
    l1_points = _feature_propagation(l1_xyz, l2_xyz, l1_points, l2_points, params['fp2'])
    VPoints.append(_voxelize(l1_xyz, l1_points, RES[1]))
    l0_points = _feature_propagation(l0_xyz, l1_xyz, l0_points, l1_points, params['fp1'])
    VPoints.append(_voxelize(l0_xyz, l0_points, RES[0]))
    point = _feature_propagation(pts, l0_xyz, None, l0_points, params['fp0'])
    x = _head(point, params['head'])
    return (x,) + tuple(VPoints)
